# baseline (speedup 1.0000x reference)
import jax
import jax.numpy as jnp
from jax.experimental import pallas as pl
from jax.experimental.pallas import tpu as pltpu

_FC = 2048
_NO = 3072
_B = 2048
_BN_EPS = 1e-5
_CC = 512
_N2 = _FC // _CC
_N3 = _NO // _CC
_NSL = 3


def _leaky(x):
    return jnp.where(x >= 0, x, 0.02 * x)


def _mlp_kernel(z_ref, w1_ref, b1_ref, g1_ref, be1_ref, m1_ref, v1_ref,
                b2_ref, g2_ref, be2_ref, m2_ref, v2_ref, b3_ref,
                w2c_ref, w3_hbm, o_hbm, h1b, h2b, w3st, ybuf, wsem, osem):
    j = pl.program_id(0)

    def w3_start(c):
        s = c % _NSL
        pltpu.make_async_copy(
            w3_hbm.at[:, pl.ds(c * _CC, _CC)],
            w3st.at[s], wsem.at[s]).start()

    for c in range(_NSL):
        @pl.when(j == c)
        def _prefetch(c=c):
            w3_start(c)

    @pl.when(j == 0)
    def _layer1():
        s1 = g1_ref[...] * jax.lax.rsqrt(v1_ref[...] + _BN_EPS)
        t1 = be1_ref[...] + (b1_ref[...] - m1_ref[...]) * s1
        zb = z_ref[...].astype(jnp.bfloat16)
        w1 = w1_ref[...].astype(jnp.bfloat16)
        h1 = jnp.dot(zb, w1, preferred_element_type=jnp.float32)
        h1b[...] = _leaky(h1 * s1 + t1).astype(jnp.bfloat16)

    @pl.when(j < _N2)
    def _layer2_chunk():
        s2 = g2_ref[...] * jax.lax.rsqrt(v2_ref[...] + _BN_EPS)
        t2 = be2_ref[...] + (b2_ref[...] - m2_ref[...]) * s2
        wc = w2c_ref[...].astype(jnp.bfloat16)
        hc = jnp.dot(h1b[...], wc, preferred_element_type=jnp.float32)
        col = pl.multiple_of(j * _CC, _CC)
        h2b[:, pl.ds(col, _CC)] = _leaky(hc * s2 + t2).astype(jnp.bfloat16)

    for c in range(_N3):
        @pl.when(j == _N2 + c)
        def _layer3_chunk(c=c):
            s = c % _NSL
            so = c % 2
            pltpu.make_async_copy(w3st.at[s], w3st.at[s], wsem.at[s]).wait()
            wc = w3st[s].astype(jnp.bfloat16)
            if c + _NSL < _N3:
                w3_start(c + _NSL)
            if c >= 2:
                pltpu.make_async_copy(ybuf.at[so], ybuf.at[so],
                                      osem.at[so]).wait()
            y = jnp.dot(h2b[...], wc, preferred_element_type=jnp.float32)
            ybuf[so] = y + b3_ref[...]
            pltpu.make_async_copy(ybuf.at[so],
                                  o_hbm.at[:, pl.ds(c * _CC, _CC)],
                                  osem.at[so]).start()
            if c == _N3 - 1:
                pltpu.make_async_copy(ybuf.at[1 - so], ybuf.at[1 - so],
                                      osem.at[1 - so]).wait()
                pltpu.make_async_copy(ybuf.at[so], ybuf.at[so],
                                      osem.at[so]).wait()


def kernel(z, l1_w, l1_b, bn1_g, bn1_b, bn1_m, bn1_v,
           l2_w, l2_b, bn2_g, bn2_b, bn2_m, bn2_v, l3_w, l3_b):
    z = z.reshape(_B, -1)
    nz = z.shape[1]

    vec = lambda a: a.reshape(1, -1)
    const = lambda shape: pl.BlockSpec(shape, lambda j: (0, 0))
    w2_idx = lambda j: (0, jnp.minimum(j, _N2 - 1))
    w3_idx = lambda j: (0, jnp.maximum(j - _N2, 0))

    return pl.pallas_call(
        _mlp_kernel,
        out_shape=jax.ShapeDtypeStruct((_B, _NO), jnp.float32),
        grid=(_N2 + _N3,),
        in_specs=[
            const((_B, nz)),
            const((nz, _FC)),
            const((1, _FC)), const((1, _FC)), const((1, _FC)),
            const((1, _FC)), const((1, _FC)),
            pl.BlockSpec((1, _CC), w2_idx),
            pl.BlockSpec((1, _CC), w2_idx),
            pl.BlockSpec((1, _CC), w2_idx),
            pl.BlockSpec((1, _CC), w2_idx),
            pl.BlockSpec((1, _CC), w2_idx),
            pl.BlockSpec((1, _CC), w3_idx),
            pl.BlockSpec((_FC, _CC), w2_idx),
            pl.BlockSpec(memory_space=pl.ANY),
        ],
        out_specs=pl.BlockSpec(memory_space=pl.ANY),
        scratch_shapes=[
            pltpu.VMEM((_B, _FC), jnp.bfloat16),
            pltpu.VMEM((_B, _FC), jnp.bfloat16),
            pltpu.VMEM((_NSL, _FC, _CC), jnp.float32),
            pltpu.VMEM((2, _B, _CC), jnp.float32),
            pltpu.SemaphoreType.DMA((_NSL,)),
            pltpu.SemaphoreType.DMA((2,)),
        ],
        compiler_params=pltpu.CompilerParams(
            dimension_semantics=("arbitrary",)),
    )(z, l1_w,
      vec(l1_b), vec(bn1_g), vec(bn1_b), vec(bn1_m), vec(bn1_v),
      vec(l2_b), vec(bn2_g), vec(bn2_b), vec(bn2_m), vec(bn2_v),
      vec(l3_b), l2_w, l3_w)


# speedup vs baseline: 1.0159x; 1.0159x over previous
import jax
import jax.numpy as jnp
from jax.experimental import pallas as pl
from jax.experimental.pallas import tpu as pltpu

_FC = 2048
_NO = 3072
_B = 2048
_BN_EPS = 1e-5
_CC = 512
_N2 = _FC // _CC
_N3 = _NO // _CC
_NSL = 3


def _leaky(x):
    return jnp.where(x >= 0, x, 0.02 * x)


def _mlp_kernel(z_ref, w1_ref, b1_ref, g1_ref, be1_ref, m1_ref, v1_ref,
                b2_ref, g2_ref, be2_ref, m2_ref, v2_ref, b3_ref,
                w2c_ref, w3_hbm, o_ref, h1b, h2b, w3st, sem):
    j = pl.program_id(0)

    def w3_start(c):
        pltpu.make_async_copy(
            w3_hbm.at[:, pl.ds(c * _CC, _CC)],
            w3st.at[c % _NSL], sem.at[c % _NSL]).start()

    for c in range(_NSL):
        @pl.when(j == c)
        def _prefetch(c=c):
            w3_start(c)

    @pl.when(j == 0)
    def _layer1():
        s1 = g1_ref[...] * jax.lax.rsqrt(v1_ref[...] + _BN_EPS)
        t1 = be1_ref[...] + (b1_ref[...] - m1_ref[...]) * s1
        zb = z_ref[...].astype(jnp.bfloat16)
        w1 = w1_ref[...].astype(jnp.bfloat16)
        h1 = jnp.dot(zb, w1, preferred_element_type=jnp.float32)
        h1b[...] = _leaky(h1 * s1 + t1).astype(jnp.bfloat16)

    @pl.when(j < _N2)
    def _layer2_chunk():
        s2 = g2_ref[...] * jax.lax.rsqrt(v2_ref[...] + _BN_EPS)
        t2 = be2_ref[...] + (b2_ref[...] - m2_ref[...]) * s2
        wc = w2c_ref[...].astype(jnp.bfloat16)
        hc = jnp.dot(h1b[...], wc, preferred_element_type=jnp.float32)
        col = pl.multiple_of(j * _CC, _CC)
        h2b[:, pl.ds(col, _CC)] = _leaky(hc * s2 + t2).astype(jnp.bfloat16)

    for c in range(_N3):
        @pl.when(j == _N2 + c)
        def _layer3_chunk(c=c):
            s = c % _NSL
            pltpu.make_async_copy(w3st.at[s], w3st.at[s], sem.at[s]).wait()
            wc = w3st[s].astype(jnp.bfloat16)
            if c + _NSL < _N3:
                w3_start(c + _NSL)
            y = jnp.dot(h2b[...], wc, preferred_element_type=jnp.float32)
            o_ref[...] = y + b3_ref[...]


def kernel(z, l1_w, l1_b, bn1_g, bn1_b, bn1_m, bn1_v,
           l2_w, l2_b, bn2_g, bn2_b, bn2_m, bn2_v, l3_w, l3_b):
    z = z.reshape(_B, -1)
    nz = z.shape[1]

    vec = lambda a: a.reshape(1, -1)
    const = lambda shape: pl.BlockSpec(shape, lambda j: (0, 0))
    w2_idx = lambda j: (0, jnp.minimum(j, _N2 - 1))
    w3_idx = lambda j: (0, jnp.maximum(j - _N2, 0))

    return pl.pallas_call(
        _mlp_kernel,
        out_shape=jax.ShapeDtypeStruct((_B, _NO), jnp.float32),
        grid=(_N2 + _N3,),
        in_specs=[
            const((_B, nz)),
            const((nz, _FC)),
            const((1, _FC)), const((1, _FC)), const((1, _FC)),
            const((1, _FC)), const((1, _FC)),
            pl.BlockSpec((1, _CC), w2_idx),
            pl.BlockSpec((1, _CC), w2_idx),
            pl.BlockSpec((1, _CC), w2_idx),
            pl.BlockSpec((1, _CC), w2_idx),
            pl.BlockSpec((1, _CC), w2_idx),
            pl.BlockSpec((1, _CC), w3_idx),
            pl.BlockSpec((_FC, _CC), w2_idx),
            pl.BlockSpec(memory_space=pl.ANY),
        ],
        out_specs=pl.BlockSpec((_B, _CC), w3_idx),
        scratch_shapes=[
            pltpu.VMEM((_B, _FC), jnp.bfloat16),
            pltpu.VMEM((_B, _FC), jnp.bfloat16),
            pltpu.VMEM((_NSL, _FC, _CC), jnp.float32),
            pltpu.SemaphoreType.DMA((_NSL,)),
        ],
        compiler_params=pltpu.CompilerParams(
            dimension_semantics=("arbitrary",)),
    )(z, l1_w,
      vec(l1_b), vec(bn1_g), vec(bn1_b), vec(bn1_m), vec(bn1_v),
      vec(l2_b), vec(bn2_g), vec(bn2_b), vec(bn2_m), vec(bn2_v),
      vec(l3_b), l2_w, l3_w)
